# baseline (speedup 1.0000x reference)
import jax
import jax.numpy as jnp
from jax import lax
from jax.experimental import pallas as pl
from jax.experimental.pallas import tpu as pltpu
from jax.experimental.pallas import tpu_sc as plsc

_N, _L, _D = 10000, 128, 128
_ROW = _L * _D
_K = 2


def _make_body(B, NW, NC):
    bpw = B // NW

    def body(idx_hbm, idxp_hbm, table_hbm, len_hbm, out_hbm, mask_hbm,
             idx_v, idxp_v, lentab_v, lens_v, buf, maskbuf, sem):
        wid = lax.axis_index("s") * NC + lax.axis_index("c")
        base = wid * bpw
        pltpu.sync_copy(idx_hbm.at[pl.ds(base, bpw)], idx_v)
        pltpu.sync_copy(idxp_hbm.at[wid], idxp_v)
        pltpu.sync_copy(len_hbm, lentab_v)
        iota = lax.iota(jnp.int32, 16)

        def lens_step(g, c):
            iv = idx_v[pl.ds(g * 16, 16)]
            lens_v[pl.ds(g * 16, 16)] = plsc.load_gather(lentab_v, [iv])
            return c
        lax.fori_loop(0, bpw // 16, lens_step, 0)

        def chunk_step(c, carry):
            gidx = idxp_v.at[pl.ds(c * 8, _K)]
            pltpu.async_copy(table_hbm.at[gidx], buf, sem).wait()
            for p in range(_K):
                b = c * _K + p
                vs = (b // 16) * 16
                lane = b - vs
                lv = lens_v[pl.ds(vs, 16)]
                len_s = jnp.sum(jnp.where(iota == lane, lv, 0))
                for j in range(_L // 16):
                    mrow = jnp.where(j * 16 + iota < len_s,
                                     jnp.int32(1), jnp.int32(0))
                    maskbuf[p, pl.ds(j * 16, 16)] = mrow
                def zrow(r, cc):
                    for j in range(_D // 16):
                        buf[p, pl.ds(r * _D + j * 16, 16)] = (
                            jnp.zeros((16,), jnp.float32))
                    return cc
                lax.fori_loop(len_s, _L, zrow, 0)
            pltpu.sync_copy(buf, out_hbm.at[pl.ds(base + c * _K, _K)])
            pltpu.sync_copy(maskbuf, mask_hbm.at[pl.ds(base + c * _K, _K)])
            return carry
        lax.fori_loop(0, bpw // _K, chunk_step, 0)

    return body


def kernel(indices, table, lengths):
    B = indices.shape[0]
    info = plsc.get_sparse_core_info()
    NC, NS = info.num_cores, info.num_subcores
    NW = NC * NS
    table2d = table.reshape(_N, _ROW)
    bpw = B // NW
    idx_pad = jnp.zeros((B // _K, 8), jnp.int32)
    idx_pad = idx_pad.at[:, :_K].set(indices.reshape(B // _K, _K))
    idx_pad = idx_pad.reshape(NW, (bpw // _K) * 8)
    run = pl.kernel(
        _make_body(B, NW, NC),
        out_type=(
            jax.ShapeDtypeStruct((B, _ROW), jnp.float32),
            jax.ShapeDtypeStruct((B, _L), jnp.int32),
        ),
        mesh=plsc.VectorSubcoreMesh(core_axis_name="c", subcore_axis_name="s"),
        compiler_params=pltpu.CompilerParams(needs_layout_passes=False),
        scratch_types=[
            pltpu.VMEM((B // NW,), jnp.int32),
            pltpu.VMEM(((bpw // _K) * 8,), jnp.int32),
            pltpu.VMEM((_N,), jnp.int32),
            pltpu.VMEM((B // NW,), jnp.int32),
            pltpu.VMEM((_K, _ROW), jnp.float32),
            pltpu.VMEM((_K, _L), jnp.int32),
            pltpu.SemaphoreType.DMA,
        ],
    )
    padded2d, mask_i = run(indices, idx_pad, table2d, lengths)
    return padded2d.reshape(B, _L, _D), mask_i.astype(jnp.bool_)


# speedup vs baseline: 1.0932x; 1.0932x over previous
import jax
import jax.numpy as jnp
from jax import lax
from jax.experimental import pallas as pl
from jax.experimental.pallas import tpu as pltpu
from jax.experimental.pallas import tpu_sc as plsc

_N, _L, _D = 10000, 128, 128
_ROW = _L * _D
_K = 2
_NBUF = 2


def _make_body(B, NW, NC):
    bpw = B // NW
    nchunks = bpw // _K

    def body(idx_hbm, idxp_hbm, table_hbm, len_hbm, out_hbm, mask_hbm,
             idx_v, idxp_v, lentab_v, lens_v, buf0, buf1, maskfull,
             gs0, gs1, ss0, ss1):
        bufs = (buf0, buf1)
        gsems = (gs0, gs1)
        ssems = (ss0, ss1)
        wid = lax.axis_index("s") * NC + lax.axis_index("c")
        base = wid * bpw
        pltpu.sync_copy(idx_hbm.at[pl.ds(base, bpw)], idx_v)
        pltpu.sync_copy(idxp_hbm.at[wid], idxp_v)
        pltpu.sync_copy(len_hbm, lentab_v)
        iota = lax.iota(jnp.int32, 16)

        def lens_step(g, c):
            iv = idx_v[pl.ds(g * 16, 16)]
            lens_v[pl.ds(g * 16, 16)] = plsc.load_gather(lentab_v, [iv])
            return c
        lax.fori_loop(0, bpw // 16, lens_step, 0)

        def len_scalar(b):
            vs = (b // 16) * 16
            lv = lens_v[pl.ds(vs, 16)]
            return jnp.sum(jnp.where(iota == b - vs, lv, 0))

        def mask_step(b, c):
            len_s = len_scalar(b)
            for j in range(_L // 16):
                maskfull[b, pl.ds(j * 16, 16)] = jnp.where(
                    j * 16 + iota < len_s, jnp.int32(1), jnp.int32(0))
            return c
        lax.fori_loop(0, bpw, mask_step, 0)

        def gather(c, b):
            gidx = idxp_v.at[pl.ds(c * 8, _K)]
            return pltpu.make_async_copy(table_hbm.at[gidx], bufs[b],
                                         gsems[b])

        def scatter(c, b):
            return pltpu.make_async_copy(
                bufs[b], out_hbm.at[pl.ds(base + c * _K, _K)], ssems[b])

        gather(0, 0).start()
        gather(1, 1).start()

        def chunk_work(c, b):
            gather(c, b).wait()
            for p in range(_K):
                bl = c * _K + p
                len_s = len_scalar(bl)

                def zrow(r, cc):
                    for j in range(_D // 16):
                        bufs[b][p, pl.ds(r * _D + j * 16, 16)] = (
                            jnp.zeros((16,), jnp.float32))
                    return cc
                lax.fori_loop(len_s, _L, zrow, 0)
            scatter(c, b).start()

            @pl.when(c + _NBUF < nchunks)
            def _():
                scatter(c, b).wait()
                gather(c + _NBUF, b).start()

        def step(i, carry):
            for b in range(_NBUF):
                chunk_work(i * _NBUF + b, b)
            return carry
        lax.fori_loop(0, nchunks // _NBUF, step, 0)

        for b in range(_NBUF):
            scatter(nchunks - _NBUF + b, b).wait()
        pltpu.sync_copy(maskfull, mask_hbm.at[pl.ds(base, bpw)])

    return body


def kernel(indices, table, lengths):
    B = indices.shape[0]
    info = plsc.get_sparse_core_info()
    NC, NS = info.num_cores, info.num_subcores
    NW = NC * NS
    table2d = table.reshape(_N, _ROW)
    bpw = B // NW
    idx_pad = jnp.zeros((B // _K, 8), jnp.int32)
    idx_pad = idx_pad.at[:, :_K].set(indices.reshape(B // _K, _K))
    idx_pad = idx_pad.reshape(NW, (bpw // _K) * 8)
    run = pl.kernel(
        _make_body(B, NW, NC),
        out_type=(
            jax.ShapeDtypeStruct((B, _ROW), jnp.float32),
            jax.ShapeDtypeStruct((B, _L), jnp.int32),
        ),
        mesh=plsc.VectorSubcoreMesh(core_axis_name="c", subcore_axis_name="s"),
        compiler_params=pltpu.CompilerParams(needs_layout_passes=False),
        scratch_types=[
            pltpu.VMEM((B // NW,), jnp.int32),
            pltpu.VMEM(((bpw // _K) * 8,), jnp.int32),
            pltpu.VMEM((_N,), jnp.int32),
            pltpu.VMEM((B // NW,), jnp.int32),
            pltpu.VMEM((_K, _ROW), jnp.float32),
            pltpu.VMEM((_K, _ROW), jnp.float32),
            pltpu.VMEM((B // NW, _L), jnp.int32),
            pltpu.SemaphoreType.DMA,
            pltpu.SemaphoreType.DMA,
            pltpu.SemaphoreType.DMA,
            pltpu.SemaphoreType.DMA,
        ],
    )
    padded2d, mask_i = run(indices, idx_pad, table2d, lengths)
    return padded2d.reshape(B, _L, _D), mask_i.astype(jnp.bool_)


# speedup vs baseline: 4.2888x; 3.9230x over previous
import jax
import jax.numpy as jnp
from jax import lax
from jax.experimental import pallas as pl
from jax.experimental.pallas import tpu as pltpu
from jax.experimental.pallas import tpu_sc as plsc

_N, _L, _D = 10000, 128, 128
_ROW = _L * _D
_K = 2
_NBUF = 2


def _make_body(B, NW, NC):
    bpw = B // NW
    nchunks = bpw // _K

    def body(idx_hbm, idxp_hbm, table_hbm, len_hbm, out_hbm, mask_hbm,
             idx_v, idxp_v, lentab_v, lens_v, buf0, buf1, maskfull,
             gs0, gs1, ss0, ss1):
        bufs = (buf0, buf1)
        gsems = (gs0, gs1)
        ssems = (ss0, ss1)
        wid = lax.axis_index("s") * NC + lax.axis_index("c")
        base = wid * bpw
        pltpu.sync_copy(idx_hbm.at[pl.ds(base, bpw)], idx_v)
        pltpu.sync_copy(idxp_hbm.at[wid], idxp_v)
        pltpu.sync_copy(len_hbm, lentab_v)
        iota = lax.iota(jnp.int32, 16)

        def lens_step(g, c):
            iv = idx_v[pl.ds(g * 16, 16)]
            lens_v[pl.ds(g * 16, 16)] = plsc.load_gather(lentab_v, [iv])
            return c
        lax.fori_loop(0, bpw // 16, lens_step, 0)

        def len_scalar(b):
            vs = (b // 16) * 16
            lv = lens_v[pl.ds(vs, 16)]
            return jnp.sum(jnp.where(iota == b - vs, lv, 0))

        def mask_step(b, c):
            len_s = len_scalar(b)
            for j in range(_L // 16):
                maskfull[b, pl.ds(j * 16, 16)] = jnp.where(
                    j * 16 + iota < len_s, jnp.int32(1), jnp.int32(0))
            return c
        lax.fori_loop(0, bpw, mask_step, 0)

        def gather(c, b):
            gidx = idxp_v.at[pl.ds(c * 8, _K)]
            return pltpu.make_async_copy(table_hbm.at[gidx], bufs[b],
                                         gsems[b])

        def scatter(c, b):
            return pltpu.make_async_copy(
                bufs[b], out_hbm.at[pl.ds(base + c * _K, _K)], ssems[b])

        gather(0, 0).start()
        gather(1, 1).start()

        def chunk_work(c, b):
            gather(c, b).wait()
            for p in range(_K):
                bl = c * _K + p
                len_s = len_scalar(bl)

                def zrow(r, cc):
                    for j in range(_D // 16):
                        bufs[b][p, r, pl.ds(j * 16, 16)] = (
                            jnp.zeros((16,), jnp.float32))
                    return cc
                lax.fori_loop(len_s, _L, zrow, 0)
            scatter(c, b).start()

            @pl.when(c + _NBUF < nchunks)
            def _():
                scatter(c, b).wait()
                gather(c + _NBUF, b).start()

        def step(i, carry):
            for b in range(_NBUF):
                chunk_work(i * _NBUF + b, b)
            return carry
        lax.fori_loop(0, nchunks // _NBUF, step, 0)

        for b in range(_NBUF):
            scatter(nchunks - _NBUF + b, b).wait()
        pltpu.sync_copy(maskfull, mask_hbm.at[pl.ds(base, bpw)])

    return body


def kernel(indices, table, lengths):
    B = indices.shape[0]
    info = plsc.get_sparse_core_info()
    NC, NS = info.num_cores, info.num_subcores
    NW = NC * NS
    bpw = B // NW
    idx_pad = jnp.zeros((B // _K, 8), jnp.int32)
    idx_pad = idx_pad.at[:, :_K].set(indices.reshape(B // _K, _K))
    idx_pad = idx_pad.reshape(NW, (bpw // _K) * 8)
    run = pl.kernel(
        _make_body(B, NW, NC),
        out_type=(
            jax.ShapeDtypeStruct((B, _L, _D), jnp.float32),
            jax.ShapeDtypeStruct((B, _L), jnp.int32),
        ),
        mesh=plsc.VectorSubcoreMesh(core_axis_name="c", subcore_axis_name="s"),
        compiler_params=pltpu.CompilerParams(needs_layout_passes=False),
        scratch_types=[
            pltpu.VMEM((B // NW,), jnp.int32),
            pltpu.VMEM(((bpw // _K) * 8,), jnp.int32),
            pltpu.VMEM((_N,), jnp.int32),
            pltpu.VMEM((B // NW,), jnp.int32),
            pltpu.VMEM((_K, _L, _D), jnp.float32),
            pltpu.VMEM((_K, _L, _D), jnp.float32),
            pltpu.VMEM((B // NW, _L), jnp.int32),
            pltpu.SemaphoreType.DMA,
            pltpu.SemaphoreType.DMA,
            pltpu.SemaphoreType.DMA,
            pltpu.SemaphoreType.DMA,
        ],
    )
    padded, mask_i = run(indices, idx_pad, table, lengths)
    return padded, mask_i.astype(jnp.bool_)


# speedup vs baseline: 4.3021x; 1.0031x over previous
import jax
import jax.numpy as jnp
from jax import lax
from jax.experimental import pallas as pl
from jax.experimental.pallas import tpu as pltpu
from jax.experimental.pallas import tpu_sc as plsc

_N, _L, _D = 10000, 128, 128
_ROW = _L * _D
_K = 1
_NBUF = 4


def _make_body(B, NW, NC):
    bpw = B // NW
    nchunks = bpw // _K

    def body(idx_hbm, idxp_hbm, table_hbm, len_hbm, out_hbm, mask_hbm,
             idx_v, idxp_v, lentab_v, lens_v, buf0, buf1, buf2, buf3,
             maskfull, gs0, gs1, gs2, gs3, ss0, ss1, ss2, ss3):
        bufs = (buf0, buf1, buf2, buf3)
        gsems = (gs0, gs1, gs2, gs3)
        ssems = (ss0, ss1, ss2, ss3)
        wid = lax.axis_index("s") * NC + lax.axis_index("c")
        base = wid * bpw
        pltpu.sync_copy(idx_hbm.at[pl.ds(base, bpw)], idx_v)
        pltpu.sync_copy(idxp_hbm.at[wid], idxp_v)
        pltpu.sync_copy(len_hbm, lentab_v)
        iota = lax.iota(jnp.int32, 16)

        def lens_step(g, c):
            iv = idx_v[pl.ds(g * 16, 16)]
            lens_v[pl.ds(g * 16, 16)] = plsc.load_gather(lentab_v, [iv])
            return c
        lax.fori_loop(0, bpw // 16, lens_step, 0)

        def len_scalar(b):
            vs = (b // 16) * 16
            lv = lens_v[pl.ds(vs, 16)]
            return jnp.sum(jnp.where(iota == b - vs, lv, 0))

        def mask_step(b, c):
            len_s = len_scalar(b)
            for j in range(_L // 16):
                maskfull[b, pl.ds(j * 16, 16)] = jnp.where(
                    j * 16 + iota < len_s, jnp.int32(1), jnp.int32(0))
            return c
        lax.fori_loop(0, bpw, mask_step, 0)

        def gather(c, b):
            gidx = idxp_v.at[pl.ds(c * 8, _K)]
            return pltpu.make_async_copy(table_hbm.at[gidx], bufs[b],
                                         gsems[b])

        def scatter(c, b):
            return pltpu.make_async_copy(
                bufs[b], out_hbm.at[pl.ds(base + c * _K, _K)], ssems[b])

        for b in range(_NBUF):
            gather(b, b).start()

        def chunk_work(c, b):
            gather(c, b).wait()
            for p in range(_K):
                bl = c * _K + p
                len_s = len_scalar(bl)

                def zrow(r, cc):
                    for j in range(_D // 16):
                        bufs[b][p, r, pl.ds(j * 16, 16)] = (
                            jnp.zeros((16,), jnp.float32))
                    return cc
                lax.fori_loop(len_s, _L, zrow, 0)
            scatter(c, b).start()

            @pl.when(c + _NBUF < nchunks)
            def _():
                scatter(c, b).wait()
                gather(c + _NBUF, b).start()

        def step(i, carry):
            for b in range(_NBUF):
                chunk_work(i * _NBUF + b, b)
            return carry
        lax.fori_loop(0, nchunks // _NBUF, step, 0)

        for b in range(_NBUF):
            scatter(nchunks - _NBUF + b, b).wait()
        pltpu.sync_copy(maskfull, mask_hbm.at[pl.ds(base, bpw)])

    return body


def kernel(indices, table, lengths):
    B = indices.shape[0]
    info = plsc.get_sparse_core_info()
    NC, NS = info.num_cores, info.num_subcores
    NW = NC * NS
    bpw = B // NW
    idx_pad = jnp.zeros((B // _K, 8), jnp.int32)
    idx_pad = idx_pad.at[:, :_K].set(indices.reshape(B // _K, _K))
    idx_pad = idx_pad.reshape(NW, (bpw // _K) * 8)
    run = pl.kernel(
        _make_body(B, NW, NC),
        out_type=(
            jax.ShapeDtypeStruct((B, _L, _D), jnp.float32),
            jax.ShapeDtypeStruct((B, _L), jnp.int32),
        ),
        mesh=plsc.VectorSubcoreMesh(core_axis_name="c", subcore_axis_name="s"),
        compiler_params=pltpu.CompilerParams(needs_layout_passes=False),
        scratch_types=[
            pltpu.VMEM((B // NW,), jnp.int32),
            pltpu.VMEM(((bpw // _K) * 8,), jnp.int32),
            pltpu.VMEM((_N,), jnp.int32),
            pltpu.VMEM((B // NW,), jnp.int32),
            pltpu.VMEM((_K, _L, _D), jnp.float32),
            pltpu.VMEM((_K, _L, _D), jnp.float32),
            pltpu.VMEM((_K, _L, _D), jnp.float32),
            pltpu.VMEM((_K, _L, _D), jnp.float32),
            pltpu.VMEM((B // NW, _L), jnp.int32),
            pltpu.SemaphoreType.DMA,
            pltpu.SemaphoreType.DMA,
            pltpu.SemaphoreType.DMA,
            pltpu.SemaphoreType.DMA,
            pltpu.SemaphoreType.DMA,
            pltpu.SemaphoreType.DMA,
            pltpu.SemaphoreType.DMA,
            pltpu.SemaphoreType.DMA,
        ],
    )
    padded, mask_i = run(indices, idx_pad, table, lengths)
    return padded, mask_i.astype(jnp.bool_)


# speedup vs baseline: 5.0020x; 1.1627x over previous
import jax
import jax.numpy as jnp
from jax import lax
from jax.experimental import pallas as pl
from jax.experimental.pallas import tpu as pltpu
from jax.experimental.pallas import tpu_sc as plsc

_N, _L, _D = 10000, 128, 128
_NBUF = 4
_G = 16


def _make_body(B, NW, NC):
    bpw = B // NW
    nchunks = bpw
    nvar = _L // _G

    def body(idx_hbm, idxp_hbm, table_hbm, len_hbm, out_hbm, mask_hbm,
             idx_v, idxp_v, lentab_v, lens_v, buf0, buf1, buf2, buf3,
             zbuf, maskfull, gs0, gs1, gs2, gs3, ss0, ss1, ss2, ss3):
        bufs = (buf0, buf1, buf2, buf3)
        gsems = (gs0, gs1, gs2, gs3)
        ssems = (ss0, ss1, ss2, ss3)
        wid = lax.axis_index("s") * NC + lax.axis_index("c")
        base = wid * bpw
        pltpu.sync_copy(idx_hbm.at[pl.ds(base, bpw)], idx_v)
        pltpu.sync_copy(idxp_hbm.at[wid], idxp_v)
        pltpu.sync_copy(len_hbm, lentab_v)
        iota = lax.iota(jnp.int32, 16)
        zero16 = jnp.zeros((16,), jnp.float32)

        def zb_step(r, c):
            for j in range(_D // 16):
                zbuf[0, r, pl.ds(j * 16, 16)] = zero16
            return c
        lax.fori_loop(0, _L, zb_step, 0)

        def lens_step(g, c):
            iv = idx_v[pl.ds(g * 16, 16)]
            lens_v[pl.ds(g * 16, 16)] = plsc.load_gather(lentab_v, [iv])
            return c
        lax.fori_loop(0, bpw // 16, lens_step, 0)

        def len_scalar(b):
            vs = (b // 16) * 16
            lv = lens_v[pl.ds(vs, 16)]
            return jnp.sum(jnp.where(iota == b - vs, lv, 0))

        def mask_step(b, c):
            len_s = len_scalar(b)
            for j in range(_L // 16):
                maskfull[b, pl.ds(j * 16, 16)] = jnp.where(
                    j * 16 + iota < len_s, jnp.int32(1), jnp.int32(0))
            return c
        lax.fori_loop(0, bpw, mask_step, 0)

        def nvalid(c):
            return (len_scalar(c) + (_G - 1)) // _G

        def gather_start(c, b):
            gidx = idxp_v.at[pl.ds(c * 8, 1)]
            nv = nvalid(c)
            for k in range(1, nvar + 1):
                @pl.when(nv == k)
                def _():
                    pltpu.make_async_copy(
                        table_hbm.at[gidx, pl.ds(0, k * _G)],
                        bufs[b].at[:, pl.ds(0, k * _G)],
                        gsems[b]).start()

        def gather_wait(c, b):
            gidx = idxp_v.at[pl.ds(c * 8, 1)]
            nv = nvalid(c)
            for k in range(1, nvar + 1):
                @pl.when(nv == k)
                def _():
                    pltpu.make_async_copy(
                        table_hbm.at[gidx, pl.ds(0, k * _G)],
                        bufs[b].at[:, pl.ds(0, k * _G)],
                        gsems[b]).wait()

        def scatter_start(c, b):
            nv = nvalid(c)
            for k in range(1, nvar + 1):
                @pl.when(nv == k)
                def _():
                    pltpu.make_async_copy(
                        bufs[b].at[:, pl.ds(0, k * _G)],
                        out_hbm.at[pl.ds(base + c, 1), pl.ds(0, k * _G)],
                        ssems[b]).start()
                    if k < nvar:
                        pltpu.make_async_copy(
                            zbuf.at[:, pl.ds(k * _G, _L - k * _G)],
                            out_hbm.at[pl.ds(base + c, 1),
                                       pl.ds(k * _G, _L - k * _G)],
                            ssems[b]).start()

        def scatter_wait(b):
            pltpu.make_async_copy(bufs[b], out_hbm.at[pl.ds(base, 1)],
                                  ssems[b]).wait()

        for b in range(_NBUF):
            gather_start(b, b)

        def chunk_work(c, b):
            gather_wait(c, b)
            len_s = len_scalar(c)
            nv16 = ((len_s + (_G - 1)) // _G) * _G

            def zrow(r, cc):
                for j in range(_D // 16):
                    bufs[b][0, r, pl.ds(j * 16, 16)] = zero16
                return cc
            lax.fori_loop(len_s, nv16, zrow, 0)

            scatter_start(c, b)

            @pl.when(c + _NBUF < nchunks)
            def _():
                scatter_wait(b)
                gather_start(c + _NBUF, b)

        def step(i, carry):
            for b in range(_NBUF):
                chunk_work(i * _NBUF + b, b)
            return carry
        lax.fori_loop(0, nchunks // _NBUF, step, 0)

        for b in range(_NBUF):
            scatter_wait(b)
        pltpu.sync_copy(maskfull, mask_hbm.at[pl.ds(base, bpw)])

    return body


def kernel(indices, table, lengths):
    B = indices.shape[0]
    info = plsc.get_sparse_core_info()
    NC, NS = info.num_cores, info.num_subcores
    NW = NC * NS
    bpw = B // NW
    idx_pad = jnp.zeros((B, 8), jnp.int32)
    idx_pad = idx_pad.at[:, 0].set(indices)
    idx_pad = idx_pad.reshape(NW, bpw * 8)
    run = pl.kernel(
        _make_body(B, NW, NC),
        out_type=(
            jax.ShapeDtypeStruct((B, _L, _D), jnp.float32),
            jax.ShapeDtypeStruct((B, _L), jnp.int32),
        ),
        mesh=plsc.VectorSubcoreMesh(core_axis_name="c", subcore_axis_name="s"),
        compiler_params=pltpu.CompilerParams(needs_layout_passes=False),
        scratch_types=[
            pltpu.VMEM((B // NW,), jnp.int32),
            pltpu.VMEM((bpw * 8,), jnp.int32),
            pltpu.VMEM((_N,), jnp.int32),
            pltpu.VMEM((B // NW,), jnp.int32),
            pltpu.VMEM((1, _L, _D), jnp.float32),
            pltpu.VMEM((1, _L, _D), jnp.float32),
            pltpu.VMEM((1, _L, _D), jnp.float32),
            pltpu.VMEM((1, _L, _D), jnp.float32),
            pltpu.VMEM((1, _L, _D), jnp.float32),
            pltpu.VMEM((B // NW, _L), jnp.int32),
            pltpu.SemaphoreType.DMA,
            pltpu.SemaphoreType.DMA,
            pltpu.SemaphoreType.DMA,
            pltpu.SemaphoreType.DMA,
            pltpu.SemaphoreType.DMA,
            pltpu.SemaphoreType.DMA,
            pltpu.SemaphoreType.DMA,
            pltpu.SemaphoreType.DMA,
        ],
    )
    padded, mask_i = run(indices, idx_pad, table, lengths)
    return padded, mask_i.astype(jnp.bool_)
